# Initial kernel scaffold; baseline (speedup 1.0000x reference)
import jax
import jax.numpy as jnp
from jax.experimental import pallas as pl


def kernel(input, weight_indices, weight_values):
    raise NotImplementedError("write your pallas kernel here")


# speedup vs baseline: 5.1893x; 5.1893x over previous
import functools

import jax
import jax.numpy as jnp
from jax import lax
from jax.experimental import pallas as pl
from jax.experimental.pallas import tpu as pltpu
from jax.experimental.pallas import tpu_sc as plsc

C = 65536
P = 65536
NNZ = 262144
NS = 16
NC = 2
LB = 16
NCHUNK = 4
EPT = NNZ // NS
GS = 128
NG = EPT // GS
RING = 4
NBLK = NG // RING
ZROWS = 256
ROWS_PER_TILE = C // NS


def _body(table, cols_h, rows_h, vals_h, out,
          cols_v, rows_v, vals_v, buf, zbuf, acc,
          g0, g1, g2, g3, s0, s1, s2, s3):
    core = lax.axis_index("c")
    tile = lax.axis_index("s")
    gsems = (g0, g1, g2, g3)
    ssems = (s0, s1, s2, s3)

    pltpu.sync_copy(cols_h.at[tile], cols_v)
    pltpu.sync_copy(rows_h.at[tile], rows_v)
    pltpu.sync_copy(vals_h.at[tile], vals_v)

    def zb_body(i, c_):
        zbuf[i, :] = jnp.zeros((LB,), jnp.float32)
        return c_
    lax.fori_loop(0, ZROWS, zb_body, 0)

    def adjust_cols(delta):
        def adj_body(i, c_):
            sl = pl.ds(i * LB, LB)
            cols_v[sl] = cols_v[sl] + delta
            return c_
        lax.fori_loop(0, EPT // LB, adj_body, 0)

    def fire_gather(g, j):
        pltpu.async_copy(table.at[cols_v.at[pl.ds(g * GS, GS)]],
                         buf.at[j], gsems[j])

    def wait_gather(j):
        pltpu.make_async_copy(table.at[pl.ds(0, GS)], buf.at[j],
                              gsems[j]).wait()

    def fire_scatter(g, j):
        pltpu.async_copy(buf.at[j], acc.at[rows_v.at[g]], ssems[j],
                         add=True)

    def wait_scatter(j):
        pltpu.make_async_copy(table.at[pl.ds(0, GS)], buf.at[j],
                              ssems[j]).wait()

    def scale(g, j):
        def sc_body(sv, c_):
            vv = vals_v[g, pl.ds(sv * LB, LB)]
            for k in range(LB):
                e = sv * LB + k
                buf[j, e, :] = buf[j, e, :] * vv[k]
            return c_
        lax.fori_loop(0, GS // LB, sc_body, 0)

    for p in range(2):
        if p == 0:
            adjust_cols(core * (2 * P))
        else:
            adjust_cols(P)
        chunk = core * 2 + p

        for zi in range(ROWS_PER_TILE // ZROWS):
            pltpu.sync_copy(
                zbuf, acc.at[pl.ds(tile * ROWS_PER_TILE + zi * ZROWS, ZROWS)])
        plsc.subcore_barrier()

        for j in range(RING):
            fire_gather(j, j)

        def blk_body(blk, c_):
            for j in range(RING):
                g = blk * RING + j
                wait_gather(j)
                scale(g, j)
                fire_scatter(g, j)
            for j in range(RING):
                wait_scatter(j)

            @pl.when(blk + 1 < NBLK)
            def _():
                for j in range(RING):
                    fire_gather((blk + 1) * RING + j, j)
            return c_
        lax.fori_loop(0, NBLK, blk_body, 0)

        plsc.subcore_barrier()
        pltpu.sync_copy(
            acc.at[pl.ds(tile * ROWS_PER_TILE, ROWS_PER_TILE)],
            out.at[chunk, pl.ds(tile * ROWS_PER_TILE, ROWS_PER_TILE)])
        plsc.subcore_barrier()


_sc_call = pl.kernel(
    _body,
    out_type=jax.ShapeDtypeStruct((NCHUNK, C, LB), jnp.float32),
    mesh=plsc.VectorSubcoreMesh(core_axis_name="c", subcore_axis_name="s"),
    scratch_types=[
        pltpu.VMEM((EPT,), jnp.int32),
        pltpu.VMEM((NG, GS), jnp.int32),
        pltpu.VMEM((NG, GS), jnp.float32),
        pltpu.VMEM((RING, GS, LB), jnp.float32),
        pltpu.VMEM((ZROWS, LB), jnp.float32),
        pltpu.VMEM_SHARED((C, LB), jnp.float32),
        pltpu.SemaphoreType.DMA,
        pltpu.SemaphoreType.DMA,
        pltpu.SemaphoreType.DMA,
        pltpu.SemaphoreType.DMA,
        pltpu.SemaphoreType.DMA,
        pltpu.SemaphoreType.DMA,
        pltpu.SemaphoreType.DMA,
        pltpu.SemaphoreType.DMA,
    ],
    compiler_params=pltpu.CompilerParams(use_tc_tiling_on_sc=False),
)


@jax.jit
def kernel(input, weight_indices, weight_values):
    rows = weight_indices[0].astype(jnp.int32)
    cols = weight_indices[1].astype(jnp.int32)
    table = input.reshape(NCHUNK, LB, P).transpose(0, 2, 1).reshape(NCHUNK * P, LB)
    out_bc = _sc_call(
        table,
        cols.reshape(NS, EPT),
        rows.reshape(NS, NG, GS),
        weight_values.reshape(NS, NG, GS),
    )
    out_right = out_bc.transpose(0, 2, 1).reshape(input.shape[0], C)
    return jnp.concatenate([input, out_right], axis=1)
